# baseline (speedup 1.0000x reference)
import functools
import jax
import jax.numpy as jnp
import numpy as np
from jax import lax
from jax.experimental import pallas as pl
from jax.experimental.pallas import tpu as pltpu
from jax.experimental.pallas import tpu_sc as plsc

WIDTH = 16
DEPTH = 4
N_PAD = 10240
EB = 4000
NW = 32
K = 80
NPW = N_PAD // NW
NPT = N_PAD // 16

_MESH = plsc.VectorSubcoreMesh(core_axis_name="c", subcore_axis_name="s")

_PERM = np.empty((256,), np.int32)
for _p in range(8):
    for _k in range(16):
        _PERM[32 * _p + 2 * _k] = 16 * (2 * _p) + _k
        _PERM[32 * _p + 2 * _k + 1] = 16 * (2 * _p + 1) + _k


def _wmlp_body(ea, k1, b1, k2, b2, k3, b3, out):
    eav = ea[...]
    a1 = b1[...]
    for j in range(eav.shape[1]):
        a1 = a1 + eav[:, j:j + 1] * k1[j:j + 1, :]
    a1 = jnp.maximum(a1, 0.0)
    a2 = jnp.maximum(
        jnp.dot(a1, k2[...], preferred_element_type=jnp.float32,
                precision=jax.lax.Precision.HIGHEST) + b2[...], 0.0)
    out[...] = jnp.dot(a2, k3[...], preferred_element_type=jnp.float32,
                       precision=jax.lax.Precision.HIGHEST) + b3[...]


def _compute_w(edge_attr, k1_W, k1_b, k2_W, k2_b, k3_W, k3_b):
    E, KIN = edge_attr.shape
    KW = k1_W.shape[1]
    OUT = k3_W.shape[1]
    grid = E // EB
    return pl.pallas_call(
        _wmlp_body,
        grid=(grid,),
        in_specs=[
            pl.BlockSpec((EB, KIN), lambda i: (i, 0)),
            pl.BlockSpec((KIN, KW), lambda i: (0, 0)),
            pl.BlockSpec((1, KW), lambda i: (0, 0)),
            pl.BlockSpec((KW, KW), lambda i: (0, 0)),
            pl.BlockSpec((1, KW), lambda i: (0, 0)),
            pl.BlockSpec((KW, OUT), lambda i: (0, 0)),
            pl.BlockSpec((1, OUT), lambda i: (0, 0)),
        ],
        out_specs=pl.BlockSpec((EB, OUT), lambda i: (i, 0)),
        out_shape=jax.ShapeDtypeStruct((E, OUT), jnp.float32),
    )(edge_attr, k1_W, k1_b.reshape(1, -1), k2_W, k2_b.reshape(1, -1),
      k3_W, k3_b.reshape(1, -1))


def _fc1_body(x, w, b, out):
    out[...] = x[...] * w[...] + b[...]


def _fc1(x_pad, fc1_W, fc1_b):
    return pl.pallas_call(
        _fc1_body,
        in_specs=[
            pl.BlockSpec((N_PAD, 1), lambda: (0, 0)),
            pl.BlockSpec((1, WIDTH), lambda: (0, 0)),
            pl.BlockSpec((1, WIDTH), lambda: (0, 0)),
        ],
        out_specs=pl.BlockSpec((N_PAD, WIDTH), lambda: (0, 0)),
        out_shape=jax.ShapeDtypeStruct((N_PAD, WIDTH), jnp.float32),
    )(x_pad, fc1_W.reshape(1, WIDTH), fc1_b.reshape(1, WIDTH))


def _fc2_body(h, w, b, out):
    out[...] = jnp.sum(h[...] * w[...], axis=1, keepdims=True) + b[...]


def _fc2(h, fc2_W, fc2_b):
    return pl.pallas_call(
        _fc2_body,
        in_specs=[
            pl.BlockSpec((N_PAD, WIDTH), lambda: (0, 0)),
            pl.BlockSpec((1, WIDTH), lambda: (0, 0)),
            pl.BlockSpec((1, 1), lambda: (0, 0)),
        ],
        out_specs=pl.BlockSpec((N_PAD, 1), lambda: (0, 0)),
        out_shape=jax.ShapeDtypeStruct((N_PAD, 1), jnp.float32),
    )(h, fc2_W.reshape(1, WIDTH), fc2_b.reshape(1, 1))


def _edge_body(with_deg, *refs):
    if with_deg:
        (src, dst, w, h, aggr2, deg2,
         srcb0, srcb1, dstb0, dstb1, dsc0, dsc1,
         xj0, xj1, wb0, wb1, msg0, msg1, onesb, zb,
         aggr_sh, h_sh, deg_sh,
         i0, i1, g0, g1, w0, w1, s0, s1, d0, d1) = refs
    else:
        (src, dst, w, h, aggr2,
         srcb0, srcb1, dstb0, dstb1, dsc0, dsc1,
         xj0, xj1, wb0, wb1, msg0, msg1, zb,
         aggr_sh, h_sh,
         i0, i1, g0, g1, w0, w1, s0, s1) = refs
        d0 = d1 = onesb = deg_sh = None

    cid = lax.axis_index("c")
    sid = lax.axis_index("s")
    wid = sid * 2 + cid
    epw = src.shape[0] // NW
    ebase = wid * epw
    nchunks = epw // K
    npairs = nchunks // 2

    zero = jnp.zeros((WIDTH,), jnp.float32)

    def zrow(j, _):
        zb[j, :] = zero
        return 0
    lax.fori_loop(0, NPT, zrow, 0, unroll=4)
    pltpu.sync_copy(zb, aggr_sh.at[pl.ds(sid * NPT, NPT)])
    pltpu.sync_copy(h.at[pl.ds(sid * NPT, NPT)], h_sh.at[pl.ds(sid * NPT, NPT)])
    if with_deg:
        pltpu.sync_copy(zb, deg_sh.at[pl.ds(sid * NPT, NPT)])
        one = jnp.ones((WIDTH,), jnp.float32)

        def orow(j, _):
            onesb[j, :] = one
            return 0
        lax.fori_loop(0, K, orow, 0, unroll=4)
    plsc.subcore_barrier()

    def fetch_idx(c, sb, db, isem):
        pltpu.async_copy(src.at[pl.ds(ebase + c * K, K)], sb, isem)
        pltpu.async_copy(dst.at[pl.ds(ebase + c * K, K)], db, isem)

    def wait_idx(sb, db, isem):
        pltpu.make_async_copy(src.at[pl.ds(0, K)], sb, isem).wait()
        pltpu.make_async_copy(dst.at[pl.ds(0, K)], db, isem).wait()

    def issue_g(c, sb, xjb, wbb, gsem, wsem):
        pltpu.async_copy(h_sh.at[sb], xjb, gsem)
        pltpu.async_copy(w.at[pl.ds(ebase + c * K, K)], wbb, wsem)

    def wait_g(xjb, wbb, sb, gsem, wsem):
        pltpu.make_async_copy(h_sh.at[sb], xjb, gsem).wait()
        pltpu.make_async_copy(w.at[pl.ds(0, K)], wbb, wsem).wait()

    def wait_s(msgb, dscb, ssem, dsem):
        pltpu.make_async_copy(msgb, aggr_sh.at[dscb], ssem).wait()
        if with_deg:
            pltpu.make_async_copy(onesb, deg_sh.at[dscb], dsem).wait()

    def copy_dsc(db, dscb):
        for t in range(K // 16):
            dscb[pl.ds(16 * t, 16)] = db[pl.ds(16 * t, 16)]

    def compute(xjb, wbb, msgb, dscb, ssem, dsem):
        def edge(e, _):
            xr = xjb[e, :]
            a0 = xr[0] * wbb[e, pl.ds(0, 16)]
            a1 = xr[1] * wbb[e, pl.ds(16, 16)]
            a2 = xr[2] * wbb[e, pl.ds(32, 16)]
            a3 = xr[3] * wbb[e, pl.ds(48, 16)]
            for i in range(4, 16, 4):
                a0 = a0 + xr[i] * wbb[e, pl.ds(i * 16, 16)]
                a1 = a1 + xr[i + 1] * wbb[e, pl.ds((i + 1) * 16, 16)]
                a2 = a2 + xr[i + 2] * wbb[e, pl.ds((i + 2) * 16, 16)]
                a3 = a3 + xr[i + 3] * wbb[e, pl.ds((i + 3) * 16, 16)]
            msgb[e, :] = (a0 + a1) + (a2 + a3)
            return 0
        lax.fori_loop(0, K, edge, 0, unroll=2)
        pltpu.async_copy(msgb, aggr_sh.at[dscb], ssem, add=True)
        if with_deg:
            pltpu.async_copy(onesb, deg_sh.at[dscb], dsem, add=True)

    fetch_idx(0, srcb0, dstb0, i0)
    fetch_idx(1, srcb1, dstb1, i1)
    wait_idx(srcb0, dstb0, i0)
    issue_g(0, srcb0, xj0, wb0, g0, w0)

    def pair(i, _):
        c0 = 2 * i
        wait_g(xj0, wb0, srcb0, g0, w0)

        @pl.when(i >= 1)
        def _():
            wait_s(msg0, dsc0, s0, d0)
        copy_dsc(dstb0, dsc0)

        @pl.when(c0 + 2 < nchunks)
        def _():
            fetch_idx(c0 + 2, srcb0, dstb0, i0)
        wait_idx(srcb1, dstb1, i1)
        issue_g(c0 + 1, srcb1, xj1, wb1, g1, w1)
        compute(xj0, wb0, msg0, dsc0, s0, d0)

        wait_g(xj1, wb1, srcb1, g1, w1)

        @pl.when(i >= 1)
        def _():
            wait_s(msg1, dsc1, s1, d1)
        copy_dsc(dstb1, dsc1)

        @pl.when(c0 + 3 < nchunks)
        def _():
            fetch_idx(c0 + 3, srcb1, dstb1, i1)

        @pl.when(c0 + 2 < nchunks)
        def _():
            wait_idx(srcb0, dstb0, i0)
            issue_g(c0 + 2, srcb0, xj0, wb0, g0, w0)
        compute(xj1, wb1, msg1, dsc1, s1, d1)
        return 0
    lax.fori_loop(0, npairs, pair, 0)

    wait_g(xj0, wb0, srcb0, g0, w0)
    wait_s(msg0, dsc0, s0, d0)
    copy_dsc(dstb0, dsc0)
    compute(xj0, wb0, msg0, dsc0, s0, d0)

    wait_s(msg0, dsc0, s0, d0)
    wait_s(msg1, dsc1, s1, d1)

    plsc.subcore_barrier()
    pltpu.sync_copy(aggr_sh.at[pl.ds(sid * NPT, NPT)],
                    aggr2.at[cid, pl.ds(sid * NPT, NPT)])
    if with_deg:
        pltpu.sync_copy(deg_sh.at[pl.ds(sid * NPT, NPT)],
                        deg2.at[cid, pl.ds(sid * NPT, NPT)])


def _edge_call(with_deg, src, dst, w, h):
    outs = [jax.ShapeDtypeStruct((2, N_PAD, WIDTH), jnp.float32)]
    scratch = [
        pltpu.VMEM((K,), jnp.int32),
        pltpu.VMEM((K,), jnp.int32),
        pltpu.VMEM((K,), jnp.int32),
        pltpu.VMEM((K,), jnp.int32),
        pltpu.VMEM((K,), jnp.int32),
        pltpu.VMEM((K,), jnp.int32),
        pltpu.VMEM((K, WIDTH), jnp.float32),
        pltpu.VMEM((K, WIDTH), jnp.float32),
        pltpu.VMEM((K, WIDTH * WIDTH), jnp.float32),
        pltpu.VMEM((K, WIDTH * WIDTH), jnp.float32),
        pltpu.VMEM((K, WIDTH), jnp.float32),
        pltpu.VMEM((K, WIDTH), jnp.float32),
    ]
    if with_deg:
        outs.append(jax.ShapeDtypeStruct((2, N_PAD, WIDTH), jnp.float32))
        scratch.append(pltpu.VMEM((K, WIDTH), jnp.float32))
    scratch.append(pltpu.VMEM((NPT, WIDTH), jnp.float32))
    scratch.append(pltpu.VMEM_SHARED((N_PAD, WIDTH), jnp.float32))
    scratch.append(pltpu.VMEM_SHARED((N_PAD, WIDTH), jnp.float32))
    if with_deg:
        scratch.append(pltpu.VMEM_SHARED((N_PAD, WIDTH), jnp.float32))
    nsem = 10 if with_deg else 8
    scratch.extend([pltpu.SemaphoreType.DMA] * nsem)
    fn = pl.kernel(
        functools.partial(_edge_body, with_deg),
        mesh=_MESH,
        out_type=tuple(outs) if with_deg else outs[0],
        scratch_types=scratch,
        compiler_params=pltpu.CompilerParams(use_tc_tiling_on_sc=False),
    )
    return fn(src, dst, w, h)


def _node_body(first, *refs):
    if first:
        (aggr2, deg2, h, rootW, cb, hn, dinv,
         p0, p1, hb, d0b, d1b, rwb, cbb, hout, dout) = refs
    else:
        (aggr2, dinv_in, h, rootW, cb, hn,
         p0, p1, hb, db, rwb, cbb, hout) = refs

    cid = lax.axis_index("c")
    sid = lax.axis_index("s")
    wid = sid * 2 + cid
    base = wid * NPW

    pltpu.sync_copy(aggr2.at[0, pl.ds(base, NPW)], p0)
    pltpu.sync_copy(aggr2.at[1, pl.ds(base, NPW)], p1)
    pltpu.sync_copy(h.at[pl.ds(base, NPW)], hb)
    if first:
        pltpu.sync_copy(deg2.at[0, pl.ds(base, NPW)], d0b)
        pltpu.sync_copy(deg2.at[1, pl.ds(base, NPW)], d1b)
    else:
        pltpu.sync_copy(dinv_in.at[pl.ds(base, NPW)], db)
    pltpu.sync_copy(rootW, rwb)
    pltpu.sync_copy(cb, cbb)

    def node(j, _):
        agg = p0[j, :] + p1[j, :]
        if first:
            dv = jnp.maximum(d0b[j, :] + d1b[j, :], 1.0)
            inv = 1.0 / dv
            dout[j, :] = inv
        else:
            inv = db[j, :]
        hr = hb[j, :]
        r0 = hr[0] * rwb[0, :]
        r1 = hr[1] * rwb[1, :]
        r2 = hr[2] * rwb[2, :]
        r3 = hr[3] * rwb[3, :]
        for i in range(4, 16, 4):
            r0 = r0 + hr[i] * rwb[i, :]
            r1 = r1 + hr[i + 1] * rwb[i + 1, :]
            r2 = r2 + hr[i + 2] * rwb[i + 2, :]
            r3 = r3 + hr[i + 3] * rwb[i + 3, :]
        hout[j, :] = jnp.maximum(agg * inv + ((r0 + r1) + (r2 + r3)) + cbb[:], 0.0)
        return 0
    lax.fori_loop(0, NPW, node, 0)

    pltpu.sync_copy(hout, hn.at[pl.ds(base, NPW)])
    if first:
        pltpu.sync_copy(dout, dinv.at[pl.ds(base, NPW)])


def _node_call(first, aggr2, degsrc, h, rootW, conv_b):
    outs = [jax.ShapeDtypeStruct((N_PAD, WIDTH), jnp.float32)]
    scratch = [
        pltpu.VMEM((NPW, WIDTH), jnp.float32),
        pltpu.VMEM((NPW, WIDTH), jnp.float32),
        pltpu.VMEM((NPW, WIDTH), jnp.float32),
    ]
    if first:
        outs.append(jax.ShapeDtypeStruct((N_PAD, WIDTH), jnp.float32))
        scratch.append(pltpu.VMEM((NPW, WIDTH), jnp.float32))
        scratch.append(pltpu.VMEM((NPW, WIDTH), jnp.float32))
    else:
        scratch.append(pltpu.VMEM((NPW, WIDTH), jnp.float32))
    scratch.append(pltpu.VMEM((WIDTH, WIDTH), jnp.float32))
    scratch.append(pltpu.VMEM((WIDTH,), jnp.float32))
    scratch.append(pltpu.VMEM((NPW, WIDTH), jnp.float32))
    if first:
        scratch.append(pltpu.VMEM((NPW, WIDTH), jnp.float32))
    fn = pl.kernel(
        functools.partial(_node_body, first),
        mesh=_MESH,
        out_type=tuple(outs) if first else outs[0],
        scratch_types=scratch,
        compiler_params=pltpu.CompilerParams(use_tc_tiling_on_sc=False),
    )
    return fn(aggr2, degsrc, h, rootW, conv_b)


def kernel(x, edge_index, edge_attr, fc1_W, fc1_b, k1_W, k1_b, k2_W, k2_b, k3_W, k3_b, root_W, conv_b, fc2_W, fc2_b):
    n = x.shape[0]
    src = edge_index[0]
    dst = edge_index[1]

    x_pad = jnp.pad(x, ((0, N_PAD - n), (0, 0)))
    h = _fc1(x_pad, fc1_W, fc1_b)
    w = _compute_w(edge_attr, k1_W, k1_b, k2_W, k2_b, k3_W, k3_b)

    aggr2, deg2 = _edge_call(True, src, dst, w, h)
    h, dinv = _node_call(True, aggr2, deg2, h, root_W, conv_b)
    for _ in range(DEPTH - 1):
        aggr2 = _edge_call(False, src, dst, w, h)
        h = _node_call(False, aggr2, dinv, h, root_W, conv_b)

    out = _fc2(h, fc2_W, fc2_b)
    return out[:n]


# speedup vs baseline: 1.5048x; 1.5048x over previous
import functools
import jax
import jax.numpy as jnp
import numpy as np
from jax import lax
from jax.experimental import pallas as pl
from jax.experimental.pallas import tpu as pltpu
from jax.experimental.pallas import tpu_sc as plsc

WIDTH = 16
DEPTH = 4
N_PAD = 10240
EB = 4000
NW = 32
K = 80
NPW = N_PAD // NW
NPT = N_PAD // 16

_MESH = plsc.VectorSubcoreMesh(core_axis_name="c", subcore_axis_name="s")

_PERM = np.empty((256,), np.int32)
for _p in range(8):
    for _k in range(16):
        _PERM[32 * _p + 2 * _k] = 16 * (2 * _p) + _k
        _PERM[32 * _p + 2 * _k + 1] = 16 * (2 * _p + 1) + _k


def _wmlp_body(ea, k1, b1, k2, b2, k3, b3, out):
    a1 = jnp.maximum(jnp.dot(ea[...], k1[...], preferred_element_type=jnp.float32) + b1[...], 0.0)
    a2 = jnp.maximum(jnp.dot(a1, k2[...], preferred_element_type=jnp.float32) + b2[...], 0.0)
    out[...] = jnp.dot(a2, k3[...], preferred_element_type=jnp.float32) + b3[...]


def _compute_w(edge_attr, k1_W, k1_b, k2_W, k2_b, k3_W, k3_b):
    E, KIN = edge_attr.shape
    KW = k1_W.shape[1]
    OUT = k3_W.shape[1]
    grid = E // EB
    return pl.pallas_call(
        _wmlp_body,
        grid=(grid,),
        in_specs=[
            pl.BlockSpec((EB, KIN), lambda i: (i, 0)),
            pl.BlockSpec((KIN, KW), lambda i: (0, 0)),
            pl.BlockSpec((1, KW), lambda i: (0, 0)),
            pl.BlockSpec((KW, KW), lambda i: (0, 0)),
            pl.BlockSpec((1, KW), lambda i: (0, 0)),
            pl.BlockSpec((KW, OUT), lambda i: (0, 0)),
            pl.BlockSpec((1, OUT), lambda i: (0, 0)),
        ],
        out_specs=pl.BlockSpec((EB, OUT), lambda i: (i, 0)),
        out_shape=jax.ShapeDtypeStruct((E, OUT), jnp.float32),
    )(edge_attr, k1_W, k1_b.reshape(1, -1), k2_W, k2_b.reshape(1, -1),
      k3_W, k3_b.reshape(1, -1))


def _fc1_body(x, w, b, out):
    out[...] = x[...] * w[...] + b[...]


def _fc1(x_pad, fc1_W, fc1_b):
    return pl.pallas_call(
        _fc1_body,
        in_specs=[
            pl.BlockSpec((N_PAD, 1), lambda: (0, 0)),
            pl.BlockSpec((1, WIDTH), lambda: (0, 0)),
            pl.BlockSpec((1, WIDTH), lambda: (0, 0)),
        ],
        out_specs=pl.BlockSpec((N_PAD, WIDTH), lambda: (0, 0)),
        out_shape=jax.ShapeDtypeStruct((N_PAD, WIDTH), jnp.float32),
    )(x_pad, fc1_W.reshape(1, WIDTH), fc1_b.reshape(1, WIDTH))


def _fc2_body(h, w, b, out):
    out[...] = jnp.sum(h[...] * w[...], axis=1, keepdims=True) + b[...]


def _fc2(h, fc2_W, fc2_b):
    return pl.pallas_call(
        _fc2_body,
        in_specs=[
            pl.BlockSpec((N_PAD, WIDTH), lambda: (0, 0)),
            pl.BlockSpec((1, WIDTH), lambda: (0, 0)),
            pl.BlockSpec((1, 1), lambda: (0, 0)),
        ],
        out_specs=pl.BlockSpec((N_PAD, 1), lambda: (0, 0)),
        out_shape=jax.ShapeDtypeStruct((N_PAD, 1), jnp.float32),
    )(h, fc2_W.reshape(1, WIDTH), fc2_b.reshape(1, 1))


def _edge_body(with_deg, *refs):
    if with_deg:
        (src, dst, w, h, aggr2, deg2,
         srcb0, srcb1, dstb0, dstb1, dsc0, dsc1,
         xj0, xj1, wb0, wb1, msg0, msg1, onesb, zb,
         aggr_sh, h_sh, deg_sh,
         i0, i1, g0, g1, w0, w1, s0, s1, d0, d1) = refs
    else:
        (src, dst, w, h, aggr2,
         srcb0, srcb1, dstb0, dstb1, dsc0, dsc1,
         xj0, xj1, wb0, wb1, msg0, msg1, zb,
         aggr_sh, h_sh,
         i0, i1, g0, g1, w0, w1, s0, s1) = refs
        d0 = d1 = onesb = deg_sh = None

    cid = lax.axis_index("c")
    sid = lax.axis_index("s")
    wid = sid * 2 + cid
    epw = src.shape[0] // NW
    ebase = wid * epw
    nchunks = epw // K
    npairs = nchunks // 2

    zero = jnp.zeros((WIDTH,), jnp.float32)

    def zrow(j, _):
        zb[j, :] = zero
        return 0
    lax.fori_loop(0, NPT, zrow, 0, unroll=4)
    pltpu.sync_copy(zb, aggr_sh.at[pl.ds(sid * NPT, NPT)])
    pltpu.sync_copy(h.at[pl.ds(sid * NPT, NPT)], h_sh.at[pl.ds(sid * NPT, NPT)])
    if with_deg:
        pltpu.sync_copy(zb, deg_sh.at[pl.ds(sid * NPT, NPT)])
        one = jnp.ones((WIDTH,), jnp.float32)

        def orow(j, _):
            onesb[j, :] = one
            return 0
        lax.fori_loop(0, K, orow, 0, unroll=4)
    plsc.subcore_barrier()

    def fetch_idx(c, sb, db, isem):
        pltpu.async_copy(src.at[pl.ds(ebase + c * K, K)], sb, isem)
        pltpu.async_copy(dst.at[pl.ds(ebase + c * K, K)], db, isem)

    def wait_idx(sb, db, isem):
        pltpu.make_async_copy(src.at[pl.ds(0, K)], sb, isem).wait()
        pltpu.make_async_copy(dst.at[pl.ds(0, K)], db, isem).wait()

    def issue_g(c, sb, xjb, wbb, gsem, wsem):
        pltpu.async_copy(h_sh.at[sb], xjb, gsem)
        pltpu.async_copy(w.at[pl.ds(ebase + c * K, K)], wbb, wsem)

    def wait_g(xjb, wbb, sb, gsem, wsem):
        pltpu.make_async_copy(h_sh.at[sb], xjb, gsem).wait()
        pltpu.make_async_copy(w.at[pl.ds(0, K)], wbb, wsem).wait()

    def wait_s(msgb, dscb, ssem, dsem):
        pltpu.make_async_copy(msgb, aggr_sh.at[dscb], ssem).wait()
        if with_deg:
            pltpu.make_async_copy(onesb, deg_sh.at[dscb], dsem).wait()

    def copy_dsc(db, dscb):
        for t in range(K // 16):
            dscb[pl.ds(16 * t, 16)] = db[pl.ds(16 * t, 16)]

    def compute(xjb, wbb, msgb, dscb, ssem, dsem):
        def edge(e, _):
            xr = xjb[e, :]
            a0 = xr[0] * wbb[e, pl.ds(0, 16)]
            a1 = xr[1] * wbb[e, pl.ds(16, 16)]
            a2 = xr[2] * wbb[e, pl.ds(32, 16)]
            a3 = xr[3] * wbb[e, pl.ds(48, 16)]
            for i in range(4, 16, 4):
                a0 = a0 + xr[i] * wbb[e, pl.ds(i * 16, 16)]
                a1 = a1 + xr[i + 1] * wbb[e, pl.ds((i + 1) * 16, 16)]
                a2 = a2 + xr[i + 2] * wbb[e, pl.ds((i + 2) * 16, 16)]
                a3 = a3 + xr[i + 3] * wbb[e, pl.ds((i + 3) * 16, 16)]
            msgb[e, :] = (a0 + a1) + (a2 + a3)
            return 0
        lax.fori_loop(0, K, edge, 0, unroll=2)
        pltpu.async_copy(msgb, aggr_sh.at[dscb], ssem, add=True)
        if with_deg:
            pltpu.async_copy(onesb, deg_sh.at[dscb], dsem, add=True)

    fetch_idx(0, srcb0, dstb0, i0)
    fetch_idx(1, srcb1, dstb1, i1)
    wait_idx(srcb0, dstb0, i0)
    issue_g(0, srcb0, xj0, wb0, g0, w0)

    def pair(i, _):
        c0 = 2 * i
        wait_g(xj0, wb0, srcb0, g0, w0)

        @pl.when(i >= 1)
        def _():
            wait_s(msg0, dsc0, s0, d0)
        copy_dsc(dstb0, dsc0)

        @pl.when(c0 + 2 < nchunks)
        def _():
            fetch_idx(c0 + 2, srcb0, dstb0, i0)
        wait_idx(srcb1, dstb1, i1)
        issue_g(c0 + 1, srcb1, xj1, wb1, g1, w1)
        compute(xj0, wb0, msg0, dsc0, s0, d0)

        wait_g(xj1, wb1, srcb1, g1, w1)

        @pl.when(i >= 1)
        def _():
            wait_s(msg1, dsc1, s1, d1)
        copy_dsc(dstb1, dsc1)

        @pl.when(c0 + 3 < nchunks)
        def _():
            fetch_idx(c0 + 3, srcb1, dstb1, i1)

        @pl.when(c0 + 2 < nchunks)
        def _():
            wait_idx(srcb0, dstb0, i0)
            issue_g(c0 + 2, srcb0, xj0, wb0, g0, w0)
        compute(xj1, wb1, msg1, dsc1, s1, d1)
        return 0
    lax.fori_loop(0, npairs, pair, 0)

    wait_g(xj0, wb0, srcb0, g0, w0)
    wait_s(msg0, dsc0, s0, d0)
    copy_dsc(dstb0, dsc0)
    compute(xj0, wb0, msg0, dsc0, s0, d0)

    wait_s(msg0, dsc0, s0, d0)
    wait_s(msg1, dsc1, s1, d1)

    plsc.subcore_barrier()
    pltpu.sync_copy(aggr_sh.at[pl.ds(sid * NPT, NPT)],
                    aggr2.at[cid, pl.ds(sid * NPT, NPT)])
    if with_deg:
        pltpu.sync_copy(deg_sh.at[pl.ds(sid * NPT, NPT)],
                        deg2.at[cid, pl.ds(sid * NPT, NPT)])


def _edge_call(with_deg, src, dst, w, h):
    outs = [jax.ShapeDtypeStruct((2, N_PAD, WIDTH), jnp.float32)]
    scratch = [
        pltpu.VMEM((K,), jnp.int32),
        pltpu.VMEM((K,), jnp.int32),
        pltpu.VMEM((K,), jnp.int32),
        pltpu.VMEM((K,), jnp.int32),
        pltpu.VMEM((K,), jnp.int32),
        pltpu.VMEM((K,), jnp.int32),
        pltpu.VMEM((K, WIDTH), jnp.float32),
        pltpu.VMEM((K, WIDTH), jnp.float32),
        pltpu.VMEM((K, WIDTH * WIDTH), jnp.float32),
        pltpu.VMEM((K, WIDTH * WIDTH), jnp.float32),
        pltpu.VMEM((K, WIDTH), jnp.float32),
        pltpu.VMEM((K, WIDTH), jnp.float32),
    ]
    if with_deg:
        outs.append(jax.ShapeDtypeStruct((2, N_PAD, WIDTH), jnp.float32))
        scratch.append(pltpu.VMEM((K, WIDTH), jnp.float32))
    scratch.append(pltpu.VMEM((NPT, WIDTH), jnp.float32))
    scratch.append(pltpu.VMEM_SHARED((N_PAD, WIDTH), jnp.float32))
    scratch.append(pltpu.VMEM_SHARED((N_PAD, WIDTH), jnp.float32))
    if with_deg:
        scratch.append(pltpu.VMEM_SHARED((N_PAD, WIDTH), jnp.float32))
    nsem = 10 if with_deg else 8
    scratch.extend([pltpu.SemaphoreType.DMA] * nsem)
    fn = pl.kernel(
        functools.partial(_edge_body, with_deg),
        mesh=_MESH,
        out_type=tuple(outs) if with_deg else outs[0],
        scratch_types=scratch,
        compiler_params=pltpu.CompilerParams(use_tc_tiling_on_sc=False),
    )
    return fn(src, dst, w, h)


def _node_body(first, *refs):
    if first:
        (aggr2, deg2, h, rootW, cb, hn, dinv,
         p0, p1, hb, d0b, d1b, rwb, cbb, hout, dout) = refs
    else:
        (aggr2, dinv_in, h, rootW, cb, hn,
         p0, p1, hb, db, rwb, cbb, hout) = refs

    cid = lax.axis_index("c")
    sid = lax.axis_index("s")
    wid = sid * 2 + cid
    base = wid * NPW

    pltpu.sync_copy(aggr2.at[0, pl.ds(base, NPW)], p0)
    pltpu.sync_copy(aggr2.at[1, pl.ds(base, NPW)], p1)
    pltpu.sync_copy(h.at[pl.ds(base, NPW)], hb)
    if first:
        pltpu.sync_copy(deg2.at[0, pl.ds(base, NPW)], d0b)
        pltpu.sync_copy(deg2.at[1, pl.ds(base, NPW)], d1b)
    else:
        pltpu.sync_copy(dinv_in.at[pl.ds(base, NPW)], db)
    pltpu.sync_copy(rootW, rwb)
    pltpu.sync_copy(cb, cbb)

    def node(j, _):
        agg = p0[j, :] + p1[j, :]
        if first:
            dv = jnp.maximum(d0b[j, :] + d1b[j, :], 1.0)
            inv = 1.0 / dv
            dout[j, :] = inv
        else:
            inv = db[j, :]
        hr = hb[j, :]
        r0 = hr[0] * rwb[0, :]
        r1 = hr[1] * rwb[1, :]
        r2 = hr[2] * rwb[2, :]
        r3 = hr[3] * rwb[3, :]
        for i in range(4, 16, 4):
            r0 = r0 + hr[i] * rwb[i, :]
            r1 = r1 + hr[i + 1] * rwb[i + 1, :]
            r2 = r2 + hr[i + 2] * rwb[i + 2, :]
            r3 = r3 + hr[i + 3] * rwb[i + 3, :]
        hout[j, :] = jnp.maximum(agg * inv + ((r0 + r1) + (r2 + r3)) + cbb[:], 0.0)
        return 0
    lax.fori_loop(0, NPW, node, 0)

    pltpu.sync_copy(hout, hn.at[pl.ds(base, NPW)])
    if first:
        pltpu.sync_copy(dout, dinv.at[pl.ds(base, NPW)])


def _node_call(first, aggr2, degsrc, h, rootW, conv_b):
    outs = [jax.ShapeDtypeStruct((N_PAD, WIDTH), jnp.float32)]
    scratch = [
        pltpu.VMEM((NPW, WIDTH), jnp.float32),
        pltpu.VMEM((NPW, WIDTH), jnp.float32),
        pltpu.VMEM((NPW, WIDTH), jnp.float32),
    ]
    if first:
        outs.append(jax.ShapeDtypeStruct((N_PAD, WIDTH), jnp.float32))
        scratch.append(pltpu.VMEM((NPW, WIDTH), jnp.float32))
        scratch.append(pltpu.VMEM((NPW, WIDTH), jnp.float32))
    else:
        scratch.append(pltpu.VMEM((NPW, WIDTH), jnp.float32))
    scratch.append(pltpu.VMEM((WIDTH, WIDTH), jnp.float32))
    scratch.append(pltpu.VMEM((WIDTH,), jnp.float32))
    scratch.append(pltpu.VMEM((NPW, WIDTH), jnp.float32))
    if first:
        scratch.append(pltpu.VMEM((NPW, WIDTH), jnp.float32))
    fn = pl.kernel(
        functools.partial(_node_body, first),
        mesh=_MESH,
        out_type=tuple(outs) if first else outs[0],
        scratch_types=scratch,
        compiler_params=pltpu.CompilerParams(use_tc_tiling_on_sc=False),
    )
    return fn(aggr2, degsrc, h, rootW, conv_b)


def kernel(x, edge_index, edge_attr, fc1_W, fc1_b, k1_W, k1_b, k2_W, k2_b, k3_W, k3_b, root_W, conv_b, fc2_W, fc2_b):
    n = x.shape[0]
    src = edge_index[0]
    dst = edge_index[1]

    x_pad = jnp.pad(x, ((0, N_PAD - n), (0, 0)))
    h = _fc1(x_pad, fc1_W, fc1_b)
    w = _compute_w(edge_attr, k1_W, k1_b, k2_W, k2_b, k3_W, k3_b)

    aggr2, deg2 = _edge_call(True, src, dst, w, h)
    h, dinv = _node_call(True, aggr2, deg2, h, root_W, conv_b)
    for _ in range(DEPTH - 1):
        aggr2 = _edge_call(False, src, dst, w, h)
        h = _node_call(False, aggr2, dinv, h, root_W, conv_b)

    out = _fc2(h, fc2_W, fc2_b)
    return out[:n]
